# baseline (speedup 1.0000x reference)
import jax
import jax.numpy as jnp
from jax.experimental import pallas as pl
from jax.experimental.pallas import tpu as pltpu

NUM_ROWS = 100000
NUM_CLASSES = 1000
BATCH = 16384

R = 64
GRID = BATCH // R


def _tc_body(idx_ref, table_ref, out_ref, sems):
    i = pl.program_id(0)
    for r in range(R):
        row = idx_ref[i * R + r]
        pltpu.make_async_copy(
            table_ref.at[pl.ds(row, 1)],
            out_ref.at[pl.ds(r, 1)],
            sems.at[r],
        ).start()
    for r in range(R):
        pltpu.make_async_copy(
            table_ref.at[pl.ds(0, 1)],
            out_ref.at[pl.ds(r, 1)],
            sems.at[r],
        ).wait()


@jax.jit
def _gather(idx, activations):
    return pl.pallas_call(
        _tc_body,
        grid_spec=pltpu.PrefetchScalarGridSpec(
            num_scalar_prefetch=1,
            grid=(GRID,),
            in_specs=[pl.BlockSpec(memory_space=pl.ANY)],
            out_specs=pl.BlockSpec((R, NUM_CLASSES), lambda i, idx_ref: (i, 0)),
            scratch_shapes=[pltpu.SemaphoreType.DMA((R,))],
        ),
        out_shape=jax.ShapeDtypeStruct((BATCH, NUM_CLASSES), jnp.float32),
    )(idx, activations)


def kernel(idx, x, activations):
    del x
    return _gather(idx.astype(jnp.int32), activations)


# speedup vs baseline: 1.2289x; 1.2289x over previous
import jax
import jax.numpy as jnp
from jax import lax
from jax.experimental import pallas as pl
from jax.experimental.pallas import tpu as pltpu
from jax.experimental.pallas import tpu_sc as plsc

NUM_ROWS = 100000
NUM_CLASSES = 1000
BATCH = 16384

S_SC = 10240
N_TC = BATCH - S_SC

NC = 2
NS = 16
NW = NC * NS
B_PER_W = S_SC // NW
CHUNK = 16
NBUF = 4
NCHUNK = B_PER_W // CHUNK
NITER = NCHUNK // NBUF

R_TC = 64
GRID_TC = N_TC // R_TC


def _sc_body(idx_hbm, table_hbm, out_hbm, idx_v, buf,
             g0, g1, g2, g3, w0, w1, w2, w3):
    wid = lax.axis_index("s") * NC + lax.axis_index("c")
    base = wid * B_PER_W

    pltpu.sync_copy(idx_hbm.at[pl.ds(base, B_PER_W)], idx_v)

    gsem = (g0, g1, g2, g3)
    wsem = (w0, w1, w2, w3)

    def issue(j, b):
        vec = idx_v[pl.ds(j * CHUNK, CHUNK)]
        for l in range(CHUNK):
            pltpu.async_copy(
                table_hbm.at[pl.ds(vec[l], 1)],
                buf.at[pl.ds(b * CHUNK + l, 1)],
                gsem[b],
            )

    def wait_gather(b):
        pltpu.make_async_copy(
            table_hbm.at[pl.ds(0, CHUNK)],
            buf.at[pl.ds(b * CHUNK, CHUNK)],
            gsem[b],
        ).wait()

    def wait_write(b):
        pltpu.make_async_copy(
            buf.at[pl.ds(b * CHUNK, CHUNK)],
            out_hbm.at[pl.ds(base, CHUNK)],
            wsem[b],
        ).wait()

    for b in range(NBUF):
        issue(b, b)

    def ring_body(t, carry):
        for b in range(NBUF):
            j = t * NBUF + b
            wait_gather(b)
            pltpu.async_copy(
                buf.at[pl.ds(b * CHUNK, CHUNK)],
                out_hbm.at[pl.ds(base + j * CHUNK, CHUNK)],
                wsem[b],
            )

            @pl.when(j + NBUF < NCHUNK)
            def _():
                wait_write(b)
                issue(j + NBUF, b)

        return carry

    lax.fori_loop(0, NITER, ring_body, 0)

    for b in range(NBUF):
        wait_write(b)


def _sc_gather(idx, activations):
    mesh = plsc.VectorSubcoreMesh(core_axis_name="c", subcore_axis_name="s")
    return pl.kernel(
        _sc_body,
        out_type=jax.ShapeDtypeStruct((S_SC, NUM_CLASSES), jnp.float32),
        mesh=mesh,
        scratch_types=[
            pltpu.VMEM((B_PER_W,), jnp.int32),
            pltpu.VMEM((NBUF * CHUNK, NUM_CLASSES), jnp.float32),
            pltpu.SemaphoreType.DMA,
            pltpu.SemaphoreType.DMA,
            pltpu.SemaphoreType.DMA,
            pltpu.SemaphoreType.DMA,
            pltpu.SemaphoreType.DMA,
            pltpu.SemaphoreType.DMA,
            pltpu.SemaphoreType.DMA,
            pltpu.SemaphoreType.DMA,
        ],
        compiler_params=pltpu.CompilerParams(use_tc_tiling_on_sc=True),
    )(idx, activations)


def _tc_body(idx_ref, table_ref, out_ref, sems):
    i = pl.program_id(0)
    for r in range(R_TC):
        row = idx_ref[i * R_TC + r]
        pltpu.make_async_copy(
            table_ref.at[pl.ds(row, 1)],
            out_ref.at[pl.ds(r, 1)],
            sems.at[r],
        ).start()
    for r in range(R_TC):
        pltpu.make_async_copy(
            table_ref.at[pl.ds(0, 1)],
            out_ref.at[pl.ds(r, 1)],
            sems.at[r],
        ).wait()


def _tc_gather(idx, activations):
    return pl.pallas_call(
        _tc_body,
        grid_spec=pltpu.PrefetchScalarGridSpec(
            num_scalar_prefetch=1,
            grid=(GRID_TC,),
            in_specs=[pl.BlockSpec(memory_space=pl.ANY)],
            out_specs=pl.BlockSpec((R_TC, NUM_CLASSES), lambda i, idx_ref: (i, 0)),
            scratch_shapes=[pltpu.SemaphoreType.DMA((R_TC,))],
        ),
        out_shape=jax.ShapeDtypeStruct((N_TC, NUM_CLASSES), jnp.float32),
    )(idx, activations)


@jax.jit
def _gather(idx, activations):
    sc_out = _sc_gather(idx[:S_SC], activations)
    tc_out = _tc_gather(idx[S_SC:], activations)
    return jnp.concatenate([sc_out, tc_out], axis=0)


def kernel(idx, x, activations):
    del x
    return _gather(idx.astype(jnp.int32), activations)


# speedup vs baseline: 1.5418x; 1.2546x over previous
import jax
import jax.numpy as jnp
from jax import lax
from jax.experimental import pallas as pl
from jax.experimental.pallas import tpu as pltpu
from jax.experimental.pallas import tpu_sc as plsc

NUM_ROWS = 100000
NUM_CLASSES = 1000
BATCH = 16384

NC = 2
NS = 16
NW = NC * NS
B_PER_W = BATCH // NW
CHUNK = 16
NBUF = 4
NCHUNK = B_PER_W // CHUNK
NITER = NCHUNK // NBUF


def _gather_body(idx_hbm, table_hbm, out_hbm, idx_v, buf,
                 g0, g1, g2, g3, w0, w1, w2, w3):
    wid = lax.axis_index("s") * NC + lax.axis_index("c")
    base = wid * B_PER_W

    pltpu.sync_copy(idx_hbm.at[pl.ds(base, B_PER_W)], idx_v)

    gsem = (g0, g1, g2, g3)
    wsem = (w0, w1, w2, w3)

    def issue(j, b):
        vec = idx_v[pl.ds(j * CHUNK, CHUNK)]
        for l in range(CHUNK):
            pltpu.async_copy(
                table_hbm.at[pl.ds(vec[l], 1)],
                buf.at[pl.ds(b * CHUNK + l, 1)],
                gsem[b],
            )

    def wait_gather(b):
        for l in range(CHUNK):
            pltpu.make_async_copy(
                table_hbm.at[pl.ds(0, 1)],
                buf.at[pl.ds(b * CHUNK + l, 1)],
                gsem[b],
            ).wait()

    def wait_write(b):
        pltpu.make_async_copy(
            buf.at[pl.ds(b * CHUNK, CHUNK)],
            out_hbm.at[pl.ds(base, CHUNK)],
            wsem[b],
        ).wait()

    for b in range(NBUF):
        issue(b, b)

    def ring_body(t, carry):
        for b in range(NBUF):
            j = t * NBUF + b
            wait_gather(b)
            pltpu.async_copy(
                buf.at[pl.ds(b * CHUNK, CHUNK)],
                out_hbm.at[pl.ds(base + j * CHUNK, CHUNK)],
                wsem[b],
            )

            @pl.when(j + NBUF < NCHUNK)
            def _():
                wait_write(b)
                issue(j + NBUF, b)

        return carry

    lax.fori_loop(0, NITER, ring_body, 0)

    for b in range(NBUF):
        wait_write(b)


@jax.jit
def _gather(idx, activations):
    mesh = plsc.VectorSubcoreMesh(core_axis_name="c", subcore_axis_name="s")
    return pl.kernel(
        _gather_body,
        out_type=jax.ShapeDtypeStruct((BATCH, NUM_CLASSES), jnp.float32),
        mesh=mesh,
        scratch_types=[
            pltpu.VMEM((B_PER_W,), jnp.int32),
            pltpu.VMEM((NBUF * CHUNK, NUM_CLASSES), jnp.float32),
            pltpu.SemaphoreType.DMA,
            pltpu.SemaphoreType.DMA,
            pltpu.SemaphoreType.DMA,
            pltpu.SemaphoreType.DMA,
            pltpu.SemaphoreType.DMA,
            pltpu.SemaphoreType.DMA,
            pltpu.SemaphoreType.DMA,
            pltpu.SemaphoreType.DMA,
        ],
        compiler_params=pltpu.CompilerParams(use_tc_tiling_on_sc=True),
    )(idx, activations)


def kernel(idx, x, activations):
    del x
    return _gather(idx.astype(jnp.int32), activations)
